# initial kernel scaffold (unmeasured)
import jax
import jax.numpy as jnp
from jax import lax
from jax.experimental import pallas as pl
from jax.experimental.pallas import tpu as pltpu


def kernel(
    x,
):
    def body(*refs):
        pass

    out_shape = jax.ShapeDtypeStruct(..., jnp.float32)
    return pl.pallas_call(body, out_shape=out_shape)(...)


# baseline (device time: 46086 ns/iter reference)
import jax
import jax.numpy as jnp
from jax import lax
from jax.experimental import pallas as pl
from jax.experimental.pallas import tpu as pltpu

NZ = 4


def kernel(x):
    _, m, n_total = x.shape
    ncol = n_total // NZ

    def body(x_ref, out_ref, send_buf, recv_buf, send_sems, recv_sems):
        my_x = lax.axis_index("x")
        my_y = lax.axis_index("y")
        my_z = lax.axis_index("z")
        left = lax.rem(my_z + (NZ - 1), NZ)
        right = lax.rem(my_z + 1, NZ)

        barrier_sem = pltpu.get_barrier_semaphore()
        for nbr in (left, right):
            pl.semaphore_signal(
                barrier_sem, inc=1,
                device_id=(my_x, my_y, nbr),
                device_id_type=pl.DeviceIdType.MESH,
            )
        pl.semaphore_wait(barrier_sem, 2)

        def local_chunk(c):
            return x_ref[0, :, pl.ds(c * ncol, ncol)]

        send_buf[...] = local_chunk(lax.rem(my_z + (NZ - 1), NZ))

        for h in range(NZ - 1):
            rdma = pltpu.make_async_remote_copy(
                src_ref=send_buf,
                dst_ref=recv_buf.at[h],
                send_sem=send_sems.at[h],
                recv_sem=recv_sems.at[h],
                device_id=(my_x, my_y, right),
                device_id_type=pl.DeviceIdType.MESH,
            )
            rdma.start()
            rdma.wait()
            c_recv = lax.rem(my_z + (2 * NZ - 2 - h), NZ)
            acc = recv_buf[h] + local_chunk(c_recv)
            if h < NZ - 2:
                send_buf[...] = acc
            else:
                out_ref[...] = acc

    return pl.pallas_call(
        body,
        out_shape=jax.ShapeDtypeStruct((m, ncol), jnp.float32),
        in_specs=[pl.BlockSpec(memory_space=pltpu.VMEM)],
        out_specs=pl.BlockSpec(memory_space=pltpu.VMEM),
        scratch_shapes=[
            pltpu.VMEM((m, ncol), jnp.float32),
            pltpu.VMEM((NZ - 1, m, ncol), jnp.float32),
            pltpu.SemaphoreType.DMA((NZ - 1,)),
            pltpu.SemaphoreType.DMA((NZ - 1,)),
        ],
        compiler_params=pltpu.CompilerParams(collective_id=0),
    )(x)


# device time: 45216 ns/iter; 1.0192x vs baseline; 1.0192x over previous
import jax
import jax.numpy as jnp
from jax import lax
from jax.experimental import pallas as pl
from jax.experimental.pallas import tpu as pltpu

NZ = 4


def kernel(x):
    _, m, n_total = x.shape
    ncol = n_total // NZ
    nh = ncol // 2

    def body(x_ref, out_ref, send_a, send_b, recv_a, recv_b,
             ss_a, rs_a, ss_b, rs_b):
        my_x = lax.axis_index("x")
        my_y = lax.axis_index("y")
        my_z = lax.axis_index("z")
        left = lax.rem(my_z + (NZ - 1), NZ)
        right = lax.rem(my_z + 1, NZ)

        barrier_sem = pltpu.get_barrier_semaphore()
        for nbr in (left, right):
            pl.semaphore_signal(
                barrier_sem, inc=1,
                device_id=(my_x, my_y, nbr),
                device_id_type=pl.DeviceIdType.MESH,
            )
        pl.semaphore_wait(barrier_sem, 2)

        def chunk_a(c):
            return x_ref[0, :, pl.ds(c * ncol, nh)]

        def chunk_b(c):
            return x_ref[0, :, pl.ds(c * ncol + nh, nh)]

        send_a[...] = chunk_a(lax.rem(my_z + (NZ - 1), NZ))
        send_b[...] = chunk_b(lax.rem(my_z + 1, NZ))

        for h in range(NZ - 1):
            rdma_a = pltpu.make_async_remote_copy(
                src_ref=send_a, dst_ref=recv_a.at[h],
                send_sem=ss_a.at[h], recv_sem=rs_a.at[h],
                device_id=(my_x, my_y, right),
                device_id_type=pl.DeviceIdType.MESH,
            )
            rdma_b = pltpu.make_async_remote_copy(
                src_ref=send_b, dst_ref=recv_b.at[h],
                send_sem=ss_b.at[h], recv_sem=rs_b.at[h],
                device_id=(my_x, my_y, left),
                device_id_type=pl.DeviceIdType.MESH,
            )
            rdma_a.start()
            rdma_b.start()
            rdma_a.wait()
            rdma_b.wait()
            c_a = lax.rem(my_z + (2 * NZ - 2 - h), NZ)
            c_b = lax.rem(my_z + 2 + h, NZ)
            acc_a = recv_a[h] + chunk_a(c_a)
            acc_b = recv_b[h] + chunk_b(c_b)
            if h < NZ - 2:
                send_a[...] = acc_a
                send_b[...] = acc_b
            else:
                out_ref[:, :nh] = acc_a
                out_ref[:, nh:] = acc_b

    return pl.pallas_call(
        body,
        out_shape=jax.ShapeDtypeStruct((m, ncol), jnp.float32),
        in_specs=[pl.BlockSpec(memory_space=pltpu.VMEM)],
        out_specs=pl.BlockSpec(memory_space=pltpu.VMEM),
        scratch_shapes=[
            pltpu.VMEM((m, nh), jnp.float32),
            pltpu.VMEM((m, nh), jnp.float32),
            pltpu.VMEM((NZ - 1, m, nh), jnp.float32),
            pltpu.VMEM((NZ - 1, m, nh), jnp.float32),
            pltpu.SemaphoreType.DMA((NZ - 1,)),
            pltpu.SemaphoreType.DMA((NZ - 1,)),
            pltpu.SemaphoreType.DMA((NZ - 1,)),
            pltpu.SemaphoreType.DMA((NZ - 1,)),
        ],
        compiler_params=pltpu.CompilerParams(collective_id=0),
    )(x)


# device time: 28699 ns/iter; 1.6058x vs baseline; 1.5755x over previous
import jax
import jax.numpy as jnp
from jax import lax
from jax.experimental import pallas as pl
from jax.experimental.pallas import tpu as pltpu

NX, NY, NZ = 2, 4, 4
MESH = pl.DeviceIdType.MESH


def kernel(x):
    _, m, n_total = x.shape
    ncol = n_total // NZ
    nrow = m // (NX * NY)
    half = NY * nrow

    def body(x_ref, out_ref, own_buf, recv_p1, recv_p2, xr_own, xr_p2,
             ss1, rs1, ss2, rs2, ss3, rs3):
        my_x = lax.axis_index("x")
        my_y = lax.axis_index("y")
        my_z = lax.axis_index("z")
        row_off = my_x * half + my_y * nrow

        barrier_sem = pltpu.get_barrier_semaphore()
        for k in range(1, NZ):
            pl.semaphore_signal(
                barrier_sem, inc=1, device_id_type=MESH,
                device_id=(my_x, my_y, lax.rem(my_z + k, NZ)))
        for k in range(1, NY):
            pl.semaphore_signal(
                barrier_sem, inc=1, device_id_type=MESH,
                device_id=(my_x, lax.rem(my_y + k, NY), my_z))
        pl.semaphore_signal(
            barrier_sem, inc=1, device_id_type=MESH,
            device_id=(1 - my_x, my_y, my_z))
        pl.semaphore_wait(barrier_sem, NZ - 1 + NY - 1 + 1)

        p1_sends = []
        for k in range(1, NZ):
            d = lax.rem(my_z + k, NZ)
            rd = pltpu.make_async_remote_copy(
                src_ref=x_ref.at[0, pl.ds(row_off, nrow),
                                 pl.ds(d * ncol, ncol)],
                dst_ref=recv_p1.at[NZ - 1 - k],
                send_sem=ss1.at[k - 1],
                recv_sem=rs1.at[NZ - 1 - k],
                device_id=(my_x, my_y, d), device_id_type=MESH,
            )
            rd.start()
            p1_sends.append(rd)
        for r in range(NZ - 1):
            pltpu.make_async_remote_copy(
                src_ref=recv_p1.at[r], dst_ref=recv_p1.at[r],
                send_sem=ss1.at[0], recv_sem=rs1.at[r],
                device_id=(my_x, my_y, my_z), device_id_type=MESH,
            ).wait_recv()
        own_buf[...] = (
            x_ref[0, pl.ds(row_off, nrow), pl.ds(my_z * ncol, ncol)]
            + recv_p1[0] + recv_p1[1] + recv_p1[2]
        )

        p2_sends = []
        for k in range(1, NY):
            p = lax.rem(my_y + k, NY)
            rd = pltpu.make_async_remote_copy(
                src_ref=own_buf,
                dst_ref=recv_p2.at[NY - 1 - k],
                send_sem=ss2.at[k - 1],
                recv_sem=rs2.at[NY - 1 - k],
                device_id=(my_x, p, my_z), device_id_type=MESH,
            )
            rd.start()
            p2_sends.append(rd)
        for r in range(NY - 1):
            pltpu.make_async_remote_copy(
                src_ref=recv_p2.at[r], dst_ref=recv_p2.at[r],
                send_sem=ss2.at[0], recv_sem=rs2.at[r],
                device_id=(my_x, my_y, my_z), device_id_type=MESH,
            ).wait_recv()

        xp = (1 - my_x, my_y, my_z)
        rd3a = pltpu.make_async_remote_copy(
            src_ref=own_buf, dst_ref=xr_own,
            send_sem=ss3.at[0], recv_sem=rs3.at[0],
            device_id=xp, device_id_type=MESH,
        )
        rd3b = pltpu.make_async_remote_copy(
            src_ref=recv_p2, dst_ref=xr_p2,
            send_sem=ss3.at[1], recv_sem=rs3.at[1],
            device_id=xp, device_id_type=MESH,
        )
        rd3a.start()
        rd3b.start()

        out_ref[pl.ds(row_off, nrow), :] = own_buf[...]
        for r in range(NY - 1):
            y_src = lax.rem(my_y + 1 + r, NY)
            out_ref[pl.ds(my_x * half + y_src * nrow, nrow), :] = recv_p2[r]

        rd3a.wait_recv()
        rd3b.wait_recv()
        other = (1 - my_x) * half
        out_ref[pl.ds(other + my_y * nrow, nrow), :] = xr_own[...]
        for r in range(NY - 1):
            y_src = lax.rem(my_y + 1 + r, NY)
            out_ref[pl.ds(other + y_src * nrow, nrow), :] = xr_p2[r]

        for rd in p1_sends + p2_sends:
            rd.wait_send()
        rd3a.wait_send()
        rd3b.wait_send()

    return pl.pallas_call(
        body,
        out_shape=jax.ShapeDtypeStruct((m, ncol), jnp.float32),
        in_specs=[pl.BlockSpec(memory_space=pltpu.VMEM)],
        out_specs=pl.BlockSpec(memory_space=pltpu.VMEM),
        scratch_shapes=[
            pltpu.VMEM((nrow, ncol), jnp.float32),
            pltpu.VMEM((NZ - 1, nrow, ncol), jnp.float32),
            pltpu.VMEM((NY - 1, nrow, ncol), jnp.float32),
            pltpu.VMEM((nrow, ncol), jnp.float32),
            pltpu.VMEM((NY - 1, nrow, ncol), jnp.float32),
            pltpu.SemaphoreType.DMA((NZ - 1,)),
            pltpu.SemaphoreType.DMA((NZ - 1,)),
            pltpu.SemaphoreType.DMA((NY - 1,)),
            pltpu.SemaphoreType.DMA((NY - 1,)),
            pltpu.SemaphoreType.DMA((2,)),
            pltpu.SemaphoreType.DMA((2,)),
        ],
        compiler_params=pltpu.CompilerParams(collective_id=0),
    )(x)


# device time: 23861 ns/iter; 1.9314x vs baseline; 1.2028x over previous
import jax
import jax.numpy as jnp
from jax import lax
from jax.experimental import pallas as pl
from jax.experimental.pallas import tpu as pltpu

NX, NY, NZ = 2, 4, 4
NH = 2
MESH = pl.DeviceIdType.MESH


def kernel(x):
    _, m, n_total = x.shape
    ncol = n_total // NZ
    nrow = m // (NX * NY)
    half = NY * nrow
    nch = ncol // NH

    def body(x_ref, out_ref, own_buf, recv_p1, recv_p2, xr_own, xr_p2,
             ss1, rs1, ss2, rs2, ss3a, rs3a, ss3b, rs3b):
        my_x = lax.axis_index("x")
        my_y = lax.axis_index("y")
        my_z = lax.axis_index("z")
        row_off = my_x * half + my_y * nrow
        xp = (1 - my_x, my_y, my_z)

        barrier_sem = pltpu.get_barrier_semaphore()
        for k in range(1, NZ):
            pl.semaphore_signal(
                barrier_sem, inc=1, device_id_type=MESH,
                device_id=(my_x, my_y, lax.rem(my_z + k, NZ)))
        for k in range(1, NY):
            pl.semaphore_signal(
                barrier_sem, inc=1, device_id_type=MESH,
                device_id=(my_x, lax.rem(my_y + k, NY), my_z))
        pl.semaphore_signal(barrier_sem, inc=1, device_id_type=MESH,
                            device_id=xp)
        pl.semaphore_wait(barrier_sem, NZ - 1 + NY - 1 + 1)

        sends = []

        for h in range(NH):
            for k in range(1, NZ):
                d = lax.rem(my_z + k, NZ)
                rd = pltpu.make_async_remote_copy(
                    src_ref=x_ref.at[0, pl.ds(row_off, nrow),
                                     pl.ds(d * ncol + h * nch, nch)],
                    dst_ref=recv_p1.at[h, NZ - 1 - k],
                    send_sem=ss1.at[h, k - 1],
                    recv_sem=rs1.at[h, NZ - 1 - k],
                    device_id=(my_x, my_y, d), device_id_type=MESH,
                )
                rd.start()
                sends.append(rd)

        rd3a = []
        for h in range(NH):
            for r in range(NZ - 1):
                pltpu.make_async_remote_copy(
                    src_ref=recv_p1.at[h, r], dst_ref=recv_p1.at[h, r],
                    send_sem=ss1.at[h, 0], recv_sem=rs1.at[h, r],
                    device_id=(my_x, my_y, my_z), device_id_type=MESH,
                ).wait_recv()
            own_buf[:, pl.ds(h * nch, nch)] = (
                x_ref[0, pl.ds(row_off, nrow),
                      pl.ds(my_z * ncol + h * nch, nch)]
                + recv_p1[h, 0] + recv_p1[h, 1] + recv_p1[h, 2]
            )
            for k in range(1, NY):
                p = lax.rem(my_y + k, NY)
                rd = pltpu.make_async_remote_copy(
                    src_ref=own_buf.at[:, pl.ds(h * nch, nch)],
                    dst_ref=recv_p2.at[NY - 1 - k, :, pl.ds(h * nch, nch)],
                    send_sem=ss2.at[h, k - 1],
                    recv_sem=rs2.at[h, NY - 1 - k],
                    device_id=(my_x, p, my_z), device_id_type=MESH,
                )
                rd.start()
                sends.append(rd)
            rd = pltpu.make_async_remote_copy(
                src_ref=own_buf.at[:, pl.ds(h * nch, nch)],
                dst_ref=xr_own.at[:, pl.ds(h * nch, nch)],
                send_sem=ss3a.at[h], recv_sem=rs3a.at[h],
                device_id=xp, device_id_type=MESH,
            )
            rd.start()
            rd3a.append(rd)
            sends.append(rd)

        rd3b = []
        for h in range(NH):
            for r in range(NY - 1):
                pltpu.make_async_remote_copy(
                    src_ref=recv_p2.at[r, :, pl.ds(h * nch, nch)],
                    dst_ref=recv_p2.at[r, :, pl.ds(h * nch, nch)],
                    send_sem=ss2.at[h, 0], recv_sem=rs2.at[h, r],
                    device_id=(my_x, my_y, my_z), device_id_type=MESH,
                ).wait_recv()
            rd = pltpu.make_async_remote_copy(
                src_ref=recv_p2.at[:, :, pl.ds(h * nch, nch)],
                dst_ref=xr_p2.at[:, :, pl.ds(h * nch, nch)],
                send_sem=ss3b.at[h], recv_sem=rs3b.at[h],
                device_id=xp, device_id_type=MESH,
            )
            rd.start()
            rd3b.append(rd)
            sends.append(rd)
            out_ref[pl.ds(row_off, nrow), pl.ds(h * nch, nch)] = (
                own_buf[:, pl.ds(h * nch, nch)])
            for r in range(NY - 1):
                y_src = lax.rem(my_y + 1 + r, NY)
                out_ref[pl.ds(my_x * half + y_src * nrow, nrow),
                        pl.ds(h * nch, nch)] = recv_p2[r, :, pl.ds(h * nch, nch)]

        other = (1 - my_x) * half
        for h in range(NH):
            rd3a[h].wait_recv()
            rd3b[h].wait_recv()
            out_ref[pl.ds(other + my_y * nrow, nrow), pl.ds(h * nch, nch)] = (
                xr_own[:, pl.ds(h * nch, nch)])
            for r in range(NY - 1):
                y_src = lax.rem(my_y + 1 + r, NY)
                out_ref[pl.ds(other + y_src * nrow, nrow),
                        pl.ds(h * nch, nch)] = xr_p2[r, :, pl.ds(h * nch, nch)]

        for rd in sends:
            rd.wait_send()

    return pl.pallas_call(
        body,
        out_shape=jax.ShapeDtypeStruct((m, ncol), jnp.float32),
        in_specs=[pl.BlockSpec(memory_space=pltpu.VMEM)],
        out_specs=pl.BlockSpec(memory_space=pltpu.VMEM),
        scratch_shapes=[
            pltpu.VMEM((nrow, ncol), jnp.float32),
            pltpu.VMEM((NH, NZ - 1, nrow, nch), jnp.float32),
            pltpu.VMEM((NY - 1, nrow, ncol), jnp.float32),
            pltpu.VMEM((nrow, ncol), jnp.float32),
            pltpu.VMEM((NY - 1, nrow, ncol), jnp.float32),
            pltpu.SemaphoreType.DMA((NH, NZ - 1)),
            pltpu.SemaphoreType.DMA((NH, NZ - 1)),
            pltpu.SemaphoreType.DMA((NH, NY - 1)),
            pltpu.SemaphoreType.DMA((NH, NY - 1)),
            pltpu.SemaphoreType.DMA((NH,)),
            pltpu.SemaphoreType.DMA((NH,)),
            pltpu.SemaphoreType.DMA((NH,)),
            pltpu.SemaphoreType.DMA((NH,)),
        ],
        compiler_params=pltpu.CompilerParams(collective_id=0),
    )(x)


# device time: 23645 ns/iter; 1.9491x vs baseline; 1.0091x over previous
import jax
import jax.numpy as jnp
from jax import lax
from jax.experimental import pallas as pl
from jax.experimental.pallas import tpu as pltpu

NX, NY, NZ = 2, 4, 4
NH = 4
MESH = pl.DeviceIdType.MESH


def kernel(x):
    _, m, n_total = x.shape
    ncol = n_total // NZ
    nrow = m // (NX * NY)
    half = NY * nrow
    nch = ncol // NH

    def body(x_ref, out_ref, own_buf, recv_p1, recv_p2, xr_own, xr_p2,
             ss1, rs1, ss2, rs2, ss3a, rs3a, ss3b, rs3b):
        my_x = lax.axis_index("x")
        my_y = lax.axis_index("y")
        my_z = lax.axis_index("z")
        row_off = my_x * half + my_y * nrow
        xp = (1 - my_x, my_y, my_z)

        barrier_sem = pltpu.get_barrier_semaphore()
        for k in range(1, NZ):
            pl.semaphore_signal(
                barrier_sem, inc=1, device_id_type=MESH,
                device_id=(my_x, my_y, lax.rem(my_z + k, NZ)))
        for k in range(1, NY):
            pl.semaphore_signal(
                barrier_sem, inc=1, device_id_type=MESH,
                device_id=(my_x, lax.rem(my_y + k, NY), my_z))
        pl.semaphore_signal(barrier_sem, inc=1, device_id_type=MESH,
                            device_id=xp)
        pl.semaphore_wait(barrier_sem, NZ - 1 + NY - 1 + 1)

        sends = []

        for h in range(NH):
            for k in range(1, NZ):
                d = lax.rem(my_z + k, NZ)
                rd = pltpu.make_async_remote_copy(
                    src_ref=x_ref.at[0, pl.ds(row_off, nrow),
                                     pl.ds(d * ncol + h * nch, nch)],
                    dst_ref=recv_p1.at[h, NZ - 1 - k],
                    send_sem=ss1.at[h, k - 1],
                    recv_sem=rs1.at[h, NZ - 1 - k],
                    device_id=(my_x, my_y, d), device_id_type=MESH,
                )
                rd.start()
                sends.append(rd)

        rd3a = []
        for h in range(NH):
            for r in range(NZ - 1):
                pltpu.make_async_remote_copy(
                    src_ref=recv_p1.at[h, r], dst_ref=recv_p1.at[h, r],
                    send_sem=ss1.at[h, 0], recv_sem=rs1.at[h, r],
                    device_id=(my_x, my_y, my_z), device_id_type=MESH,
                ).wait_recv()
            own_buf[:, pl.ds(h * nch, nch)] = (
                x_ref[0, pl.ds(row_off, nrow),
                      pl.ds(my_z * ncol + h * nch, nch)]
                + recv_p1[h, 0] + recv_p1[h, 1] + recv_p1[h, 2]
            )
            for k in range(1, NY):
                p = lax.rem(my_y + k, NY)
                rd = pltpu.make_async_remote_copy(
                    src_ref=own_buf.at[:, pl.ds(h * nch, nch)],
                    dst_ref=recv_p2.at[NY - 1 - k, :, pl.ds(h * nch, nch)],
                    send_sem=ss2.at[h, k - 1],
                    recv_sem=rs2.at[h, NY - 1 - k],
                    device_id=(my_x, p, my_z), device_id_type=MESH,
                )
                rd.start()
                sends.append(rd)
            rd = pltpu.make_async_remote_copy(
                src_ref=own_buf.at[:, pl.ds(h * nch, nch)],
                dst_ref=xr_own.at[:, pl.ds(h * nch, nch)],
                send_sem=ss3a.at[h], recv_sem=rs3a.at[h],
                device_id=xp, device_id_type=MESH,
            )
            rd.start()
            rd3a.append(rd)
            sends.append(rd)

        rd3b = []
        for h in range(NH):
            for r in range(NY - 1):
                pltpu.make_async_remote_copy(
                    src_ref=recv_p2.at[r, :, pl.ds(h * nch, nch)],
                    dst_ref=recv_p2.at[r, :, pl.ds(h * nch, nch)],
                    send_sem=ss2.at[h, 0], recv_sem=rs2.at[h, r],
                    device_id=(my_x, my_y, my_z), device_id_type=MESH,
                ).wait_recv()
            rd = pltpu.make_async_remote_copy(
                src_ref=recv_p2.at[:, :, pl.ds(h * nch, nch)],
                dst_ref=xr_p2.at[:, :, pl.ds(h * nch, nch)],
                send_sem=ss3b.at[h], recv_sem=rs3b.at[h],
                device_id=xp, device_id_type=MESH,
            )
            rd.start()
            rd3b.append(rd)
            sends.append(rd)
            out_ref[pl.ds(row_off, nrow), pl.ds(h * nch, nch)] = (
                own_buf[:, pl.ds(h * nch, nch)])
            for r in range(NY - 1):
                y_src = lax.rem(my_y + 1 + r, NY)
                out_ref[pl.ds(my_x * half + y_src * nrow, nrow),
                        pl.ds(h * nch, nch)] = recv_p2[r, :, pl.ds(h * nch, nch)]

        other = (1 - my_x) * half
        for h in range(NH):
            rd3a[h].wait_recv()
            rd3b[h].wait_recv()
            out_ref[pl.ds(other + my_y * nrow, nrow), pl.ds(h * nch, nch)] = (
                xr_own[:, pl.ds(h * nch, nch)])
            for r in range(NY - 1):
                y_src = lax.rem(my_y + 1 + r, NY)
                out_ref[pl.ds(other + y_src * nrow, nrow),
                        pl.ds(h * nch, nch)] = xr_p2[r, :, pl.ds(h * nch, nch)]

        for rd in sends:
            rd.wait_send()

    return pl.pallas_call(
        body,
        out_shape=jax.ShapeDtypeStruct((m, ncol), jnp.float32),
        in_specs=[pl.BlockSpec(memory_space=pltpu.VMEM)],
        out_specs=pl.BlockSpec(memory_space=pltpu.VMEM),
        scratch_shapes=[
            pltpu.VMEM((nrow, ncol), jnp.float32),
            pltpu.VMEM((NH, NZ - 1, nrow, nch), jnp.float32),
            pltpu.VMEM((NY - 1, nrow, ncol), jnp.float32),
            pltpu.VMEM((nrow, ncol), jnp.float32),
            pltpu.VMEM((NY - 1, nrow, ncol), jnp.float32),
            pltpu.SemaphoreType.DMA((NH, NZ - 1)),
            pltpu.SemaphoreType.DMA((NH, NZ - 1)),
            pltpu.SemaphoreType.DMA((NH, NY - 1)),
            pltpu.SemaphoreType.DMA((NH, NY - 1)),
            pltpu.SemaphoreType.DMA((NH,)),
            pltpu.SemaphoreType.DMA((NH,)),
            pltpu.SemaphoreType.DMA((NH,)),
            pltpu.SemaphoreType.DMA((NH,)),
        ],
        compiler_params=pltpu.CompilerParams(collective_id=0),
    )(x)
